# baseline (speedup 1.0000x reference)
import jax
import jax.numpy as jnp
from jax.experimental import pallas as pl

_B_BLK = 128


def _body(x_ref, states_ref, len_ref):
    x = x_ref[...]
    states_ref[...] = jnp.transpose(x, (1, 0, 2))
    rows = jnp.sum(x, axis=2)
    len_ref[...] = jnp.sum(rows != 0.0, axis=0, dtype=jnp.int32)[None, :]


def kernel(batch):
    S, B, D = batch.shape
    grid = (B // _B_BLK,)
    states, lengths = pl.pallas_call(
        _body,
        grid=grid,
        in_specs=[pl.BlockSpec((S, _B_BLK, D), lambda i: (0, i, 0))],
        out_specs=[
            pl.BlockSpec((_B_BLK, S, D), lambda i: (i, 0, 0)),
            pl.BlockSpec((1, _B_BLK), lambda i: (0, i)),
        ],
        out_shape=[
            jax.ShapeDtypeStruct((B, S, D), batch.dtype),
            jax.ShapeDtypeStruct((1, B), jnp.int32),
        ],
    )(batch)
    return states, lengths.reshape(B)


# speedup vs baseline: 1.0030x; 1.0030x over previous
import jax
import jax.numpy as jnp
from jax.experimental import pallas as pl
from jax.experimental.pallas import tpu as pltpu

_S_BLK = 8


def _body(x_ref, states_ref, len_ref, sems):
    i = pl.program_id(0)
    for k in range(_S_BLK):
        pltpu.make_async_copy(
            x_ref.at[k],
            states_ref.at[:, i * _S_BLK + k, :],
            sems.at[k],
        ).start()
    rows = jnp.sum(x_ref[...], axis=2)
    cnt = jnp.sum(rows != 0.0, axis=0, dtype=jnp.int32)[None, :]

    @pl.when(i == 0)
    def _():
        len_ref[...] = jnp.zeros_like(len_ref)

    len_ref[...] += cnt
    for k in range(_S_BLK):
        pltpu.make_async_copy(
            x_ref.at[k],
            states_ref.at[:, i * _S_BLK + k, :],
            sems.at[k],
        ).wait()


def kernel(batch):
    S, B, D = batch.shape
    states, lengths = pl.pallas_call(
        _body,
        grid=(S // _S_BLK,),
        in_specs=[pl.BlockSpec((_S_BLK, B, D), lambda i: (i, 0, 0))],
        out_specs=[
            pl.BlockSpec(memory_space=pltpu.MemorySpace.HBM),
            pl.BlockSpec((1, B), lambda i: (0, 0)),
        ],
        out_shape=[
            jax.ShapeDtypeStruct((B, S, D), batch.dtype),
            jax.ShapeDtypeStruct((1, B), jnp.int32),
        ],
        scratch_shapes=[pltpu.SemaphoreType.DMA((_S_BLK,))],
    )(batch)
    return states, lengths.reshape(B)
